# baseline (speedup 1.0000x reference)
import functools

import jax
import jax.numpy as jnp
import numpy as np
from jax import lax
from jax.experimental import pallas as pl
from jax.experimental.pallas import tpu as pltpu
from jax.experimental.pallas import tpu_sc as plsc

_PX, _PY = 2, 2
_K = _PX * _PY


def _np_threefry2x32(k0, k1, x0, x1):
    rot = ((13, 15, 26, 6), (17, 29, 16, 24))
    ks = (np.uint32(k0), np.uint32(k1),
          np.uint32(k0) ^ np.uint32(k1) ^ np.uint32(0x1BD11BDA))
    x0 = (x0 + ks[0]).astype(np.uint32)
    x1 = (x1 + ks[1]).astype(np.uint32)

    def rotl(v, r):
        return ((v << np.uint32(r)) | (v >> np.uint32(32 - r))).astype(np.uint32)

    inject = ((1, 2, 1), (2, 0, 2), (0, 1, 3), (1, 2, 4), (2, 0, 5))
    for i, (a, b, c) in enumerate(inject):
        for r in rot[i % 2]:
            x0 = (x0 + x1).astype(np.uint32)
            x1 = rotl(x1, r)
            x1 = x1 ^ x0
        x0 = (x0 + ks[a]).astype(np.uint32)
        x1 = (x1 + ks[b] + np.uint32(c)).astype(np.uint32)
    return x0, x1


@functools.lru_cache(maxsize=1)
def _gumbel_exp_weights_t(seed, rows, k):
    size = rows * k
    k0 = np.uint32(np.int64(seed) >> 32 & 0xFFFFFFFF)
    k1 = np.uint32(np.int64(seed) & 0xFFFFFFFF)
    idx = np.arange(size, dtype=np.uint64)
    hi = (idx >> np.uint64(32)).astype(np.uint32)
    lo = (idx & np.uint64(0xFFFFFFFF)).astype(np.uint32)
    y0, y1 = _np_threefry2x32(k0, k1, hi, lo)
    bits = y0 ^ y1
    float_bits = (bits >> np.uint32(9)) | np.uint32(0x3F800000)
    floats = float_bits.view(np.float32) - np.float32(1.0)
    tiny = np.float32(np.finfo(np.float32).tiny)
    u = np.maximum(tiny, floats * (np.float32(1.0) - tiny) + tiny)
    w = (np.float32(-1.0) / np.log(u)).astype(np.float32)
    return np.ascontiguousarray(w.reshape(rows, k).T)


def _make_sc_sampler(C, NM, L, q0=0, ncall=None):
    LC = 128
    NW = 32
    NCHUNK = ncall if ncall is not None else NM // LC
    ROWS_C = LC * C
    mesh = plsc.VectorSubcoreMesh(core_axis_name="c", subcore_axis_name="s")

    @functools.partial(
        pl.kernel,
        out_type=jax.ShapeDtypeStruct((C, NCHUNK * LC), jnp.float32),
        mesh=mesh,
        compiler_params=pltpu.CompilerParams(needs_layout_passes=False),
        scratch_types=[
            pltpu.VMEM((_K, ROWS_C), jnp.float32),
            pltpu.VMEM((_K * ROWS_C,), jnp.float32),
            pltpu.VMEM((C, LC), jnp.float32),
        ],
    )
    def sc_sampler(et_hbm, w_hbm, zt_hbm, etv, wv, zbuf):
        wid = lax.axis_index("s") * 2 + lax.axis_index("c")
        lane = lax.iota(jnp.int32, 16)
        nphase = C // 16
        for t in range(-(-NCHUNK // NW)):
            q = wid + NW * t

            @pl.when(q < NCHUNK)
            def _():
                col0 = pl.multiple_of(q * LC, LC)
                gq = q + q0
                l0 = pl.multiple_of(gq * ROWS_C, ROWS_C)
                pltpu.sync_copy(et_hbm.at[:, pl.ds(l0, ROWS_C)], etv)
                pltpu.sync_copy(
                    w_hbm.at[pl.ds(pl.multiple_of(gq * (_K * ROWS_C), ROWS_C),
                                   _K * ROWS_C)],
                    wv,
                )

                def body(d, carry):
                    dv = jnp.broadcast_to(d, (16,))
                    base = d * C
                    for p in range(nphase):
                        off = base + p * 16
                        mx = (etv[0, pl.ds(off, 16)] + 1e-20) * wv[pl.ds(off, 16)]
                        z = jnp.zeros((16,), jnp.float32)
                        for k in (1, 2, 3):
                            mk = (etv[k, pl.ds(off, 16)] + 1e-20) * wv[
                                pl.ds(k * ROWS_C + off, 16)]
                            z = jnp.where(mk > mx, jnp.float32(k), z)
                            mx = jnp.maximum(mk, mx)
                        cc = p * 16 + lane
                        plsc.store_scatter(zbuf, [cc, dv], z)
                    return carry

                lax.fori_loop(0, LC, body, 0)
                pltpu.sync_copy(zbuf, zt_hbm.at[:, pl.ds(col0, LC)])

    return sc_sampler


def _expand_body(s_ref, z_ref, out_ref):
    bsz, csz, _, m = s_ref.shape
    ci = jax.lax.broadcasted_iota(jnp.int32, (csz, 2 * m), 1)
    cidx = ci // 2
    par = (ci % 2).astype(jnp.float32)
    for si in range(8):
        zz = jnp.take_along_axis(z_ref[:, si, :], cidx, axis=1)
        m0 = zz == par
        m1 = zz == par + 2.0
        for b in range(bsz):
            ss = jnp.take_along_axis(s_ref[b, :, si, :], cidx, axis=1)
            out_ref[b, :, 2 * si, :] = jnp.where(m0, ss, 0.0)
            out_ref[b, :, 2 * si + 1, :] = jnp.where(m1, ss, 0.0)


@jax.jit
def kernel(s, etas):
    B, C, n, m = s.shape
    nm = n * m
    L = nm * C

    wt = _gumbel_exp_weights_t(42, L, _K)

    et = jnp.transpose(etas)

    G = 14
    GH = G // 2
    NCH = nm // 128
    NCH1 = 49

    rows_c = 128 * C
    w_chunked = np.ascontiguousarray(
        wt.reshape(_K, L // rows_c, rows_c).transpose(1, 0, 2)
    ).reshape(-1)
    wj = jnp.asarray(w_chunked)

    zeta1 = _make_sc_sampler(C, nm, L, q0=0, ncall=NCH1)(et, wj)
    zeta2 = _make_sc_sampler(C, nm, L, q0=NCH1, ncall=NCH - NCH1)(et, wj)
    zh = n // 2
    z3a = zeta1.reshape(C, zh, m)
    z3b = zeta2.reshape(C, zh, m)

    def expand_call(zeta3, row0, prev):
        extra_in, extra_specs, aliases = (), (), {}
        if prev is not None:
            extra_in = (prev,)
            extra_specs = (pl.BlockSpec(memory_space=pl.ANY),)
            aliases = {2: 0}

        def body(s_ref, z_ref, *rest):
            _expand_body(s_ref, z_ref, rest[-1])

        return pl.pallas_call(
            body,
            grid=(GH,),
            in_specs=[
                pl.BlockSpec((B, C, 8, m), lambda g: (0, 0, g + row0, 0)),
                pl.BlockSpec((C, 8, m), lambda g: (0, g, 0)),
                *extra_specs,
            ],
            out_specs=pl.BlockSpec(
                (B, C, 16, _PY * m), lambda g: (0, 0, g + row0, 0)
            ),
            out_shape=jax.ShapeDtypeStruct((B, C, _PX * n, _PY * m), jnp.float32),
            input_output_aliases=aliases,
        )(s, zeta3, *extra_in)

    out1 = expand_call(z3a, 0, None)
    out = expand_call(z3b, GH, out1)
    return out
